# baseline (speedup 1.0000x reference)
import functools

import jax
import jax.numpy as jnp
import numpy as np
from jax import lax
from jax.experimental import pallas as pl
from jax.experimental.pallas import tpu as pltpu
from jax.experimental.pallas import tpu_sc as plsc

D = 128
DR = 128
H = 4
DH = D // H
FF = 256
KK = 8
TEMP = 0.1
_SQRT_DH = np.sqrt(DH).astype(np.float32)
_SQRT_HALF = np.sqrt(0.5).astype(np.float32)


def _ln(x, g, b, eps=1e-5):
    m = jnp.mean(x, axis=-1, keepdims=True)
    v = jnp.mean((x - m) ** 2, axis=-1, keepdims=True)
    return (x - m) / jnp.sqrt(v + eps) * g + b


def _softmax(x):
    m = jnp.max(x, axis=-1, keepdims=True)
    e = jnp.exp(x - m)
    return e / jnp.sum(e, axis=-1, keepdims=True)


def _gelu(x):
    return 0.5 * x * (1.0 + lax.erf(x * _SQRT_HALF))


def _bdot(a, w_bf16):
    return jnp.dot(a.astype(jnp.bfloat16), w_bf16,
                   preferred_element_type=jnp.float32)


def _encoder_kernel(L, SB, n_layers, h_ref, *refs):
    it = iter(refs)
    layers = [[next(it) for _ in range(12)] for _ in range(n_layers)]
    (wp_ref, bp_ref, gp_ref, bpl_ref, wv1_ref, bv1_ref, gv_ref, bvl_ref,
     wv2_ref, bv2_ref, swv_ref, oswv_ref) = [next(it) for _ in range(12)]
    out_ref = next(it)

    x = h_ref[...]
    for (wqkv_ref, bqkv_ref, wo_ref, bo_ref, g1_ref, b1_ref, w1_ref,
         bff1_ref, w2_ref, bff2_ref, g2_ref, b2_ref) in layers:
        qkv = _bdot(x, wqkv_ref[...]) + bqkv_ref[...]
        qb = qkv[:, :D].astype(jnp.bfloat16)
        kb = qkv[:, D:2 * D].astype(jnp.bfloat16)
        vb = qkv[:, 2 * D:].astype(jnp.bfloat16)
        outs = []
        for s in range(SB):
            r0 = s * L
            heads = []
            for hh in range(H):
                sl = slice(hh * DH, (hh + 1) * DH)
                sc = lax.dot_general(qb[r0:r0 + L, sl], kb[r0:r0 + L, sl],
                                     (((1,), (1,)), ((), ())),
                                     preferred_element_type=jnp.float32)
                a = _softmax(sc / _SQRT_DH)
                heads.append(jnp.dot(a.astype(jnp.bfloat16), vb[r0:r0 + L, sl],
                                     preferred_element_type=jnp.float32))
            outs.append(jnp.concatenate(heads, axis=1))
        o = jnp.concatenate(outs, axis=0)
        o = _bdot(o, wo_ref[...]) + bo_ref[...]
        x = _ln(x + o, g1_ref[...], b1_ref[...])
        h = _bdot(x, w1_ref[...]) + bff1_ref[...]
        h = _gelu(h)
        h = _bdot(h, w2_ref[...]) + bff2_ref[...]
        x = _ln(x + h, g2_ref[...], b2_ref[...])

    cls = jnp.concatenate([x[s * L:s * L + 1, :] for s in range(SB)], axis=0)
    qr = _ln(_bdot(cls, wp_ref[...]) + bp_ref[...], gp_ref[...], bpl_ref[...])

    xm = jnp.concatenate(
        [jnp.mean(h_ref[s * L + 1:(s + 1) * L, :], axis=0, keepdims=True)
         for s in range(SB)], axis=0)
    hv = _bdot(xm, wv1_ref[...]) + bv1_ref[...]
    hv = _gelu(_ln(hv, gv_ref[...], bvl_ref[...]))
    qv = _bdot(hv, wv2_ref[...]) + bv2_ref[...]

    out_ref[...] = swv_ref[...] * qr + oswv_ref[...] * qv


def _topk_kernel(n_keys, qk_ref, mk_ref, w_ref, ti_ref, ts_scr):
    qk = qk_ref[...]
    mk = mk_ref[...]
    dot = lax.dot_general(qk, mk, (((1,), (1,)), ((), ())),
                          preferred_element_type=jnp.float32,
                          precision=lax.Precision.HIGHEST)
    nq = jnp.sum(qk * qk, axis=1, keepdims=True)
    ones = jnp.ones((1, mk.shape[1]), jnp.float32)
    nk = lax.dot_general(ones, mk * mk, (((1,), (1,)), ((), ())),
                         preferred_element_type=jnp.float32,
                         precision=lax.Precision.HIGHEST)
    d = nq + nk - 2.0 * dot
    sim = -d / TEMP
    iota = lax.broadcasted_iota(jnp.int32, sim.shape, 1)
    for j in range(KK):
        m = jnp.max(sim, axis=1, keepdims=True)
        cand = jnp.where(sim == m, iota, jnp.int32(2 ** 30))
        idx = jnp.min(cand, axis=1, keepdims=True)
        ts_scr[:, j:j + 1] = m
        ti_ref[:, j:j + 1] = idx
        sim = jnp.where(iota == idx, -jnp.inf, sim)
    w_ref[...] = _softmax(ts_scr[...])


def _sc_gather(memory_values, idx):
    n_rows, sl, dd = memory_values.shape
    b = idx.shape[0]
    info = plsc.get_sparse_core_info()
    nc, ns = info.num_cores, info.num_subcores
    nw = nc * ns
    b_per_w = b // nw
    mesh = plsc.VectorSubcoreMesh(core_axis_name="c", subcore_axis_name="s")

    @functools.partial(
        pl.kernel, mesh=mesh,
        out_type=jax.ShapeDtypeStruct((b, sl, dd), jnp.float32),
        scratch_types=[
            pltpu.VMEM((b_per_w,), jnp.int32),
            pltpu.VMEM((b_per_w, sl, dd), jnp.float32),
            pltpu.SemaphoreType.DMA,
        ],
    )
    def gather(mv_hbm, idx_hbm, out_hbm, idx_v, rows_v, sem):
        wid = lax.axis_index("s") * nc + lax.axis_index("c")
        base = wid * b_per_w
        pltpu.sync_copy(idx_hbm.at[pl.ds(base, b_per_w)], idx_v)
        pltpu.async_copy(mv_hbm.at[idx_v], rows_v, sem).wait()
        pltpu.sync_copy(rows_v, out_hbm.at[pl.ds(base, b_per_w)])

    return gather(memory_values, idx)


def kernel(query, memory_keys, memory_values, params):
    p = params
    b, lq, _ = query.shape
    L = lq + 1
    n_layers = len(p['layers'])

    cls = jnp.broadcast_to(p['cls'], (b, 1, D))
    h0 = jnp.concatenate([cls, query], axis=1)

    def wt(w):
        return w.T.astype(jnp.bfloat16)

    def row(v):
        return v.reshape(1, -1)

    wrefs = []
    for lp in p['layers']:
        wrefs += [wt(lp['Wqkv']), row(lp['bqkv']), wt(lp['Wo']), row(lp['bo']),
                  row(lp['g1']), row(lp['b1']), wt(lp['W1']), row(lp['bff1']),
                  wt(lp['W2']), row(lp['bff2']), row(lp['g2']), row(lp['b2'])]
    sw = p['sw']
    swv = jnp.broadcast_to(sw.reshape(1, 1), (1, D)).astype(jnp.float32)
    oswv = jnp.broadcast_to((1.0 - sw).reshape(1, 1), (1, D)).astype(jnp.float32)
    wrefs += [wt(p['Wp']), row(p['bp']), row(p['gp']), row(p['bpl']),
              wt(p['Wv1']), row(p['bv1']), row(p['gv']), row(p['bvl']),
              wt(p['Wv2']), row(p['bv2']), swv, oswv]

    const_spec = [pl.BlockSpec(x.shape, lambda i, nd=x.ndim: (0,) * nd)
                  for x in wrefs]
    SB = 16
    hflat = h0.reshape(b * L, D)
    qk = pl.pallas_call(
        functools.partial(_encoder_kernel, L, SB, n_layers),
        grid=(b // SB,),
        in_specs=[pl.BlockSpec((SB * L, D), lambda i: (i, 0))] + const_spec,
        out_specs=pl.BlockSpec((SB, D), lambda i: (i, 0)),
        out_shape=jax.ShapeDtypeStruct((b, D), jnp.float32),
    )(hflat, *wrefs)

    n_keys = memory_keys.shape[0]
    w, ti = pl.pallas_call(
        functools.partial(_topk_kernel, n_keys),
        out_shape=[jax.ShapeDtypeStruct((b, KK), jnp.float32),
                   jax.ShapeDtypeStruct((b, KK), jnp.int32)],
        scratch_shapes=[pltpu.VMEM((b, KK), jnp.float32)],
    )(qk, memory_keys)

    refs_flat = _sc_gather(memory_values, ti.reshape(b * KK))
    refs = refs_flat.reshape(b, KK, *memory_values.shape[1:])
    return refs, w


# speedup vs baseline: 1.0011x; 1.0011x over previous
import functools

import jax
import jax.numpy as jnp
import numpy as np
from jax import lax
from jax.experimental import pallas as pl
from jax.experimental.pallas import tpu as pltpu
from jax.experimental.pallas import tpu_sc as plsc

D = 128
DR = 128
H = 4
DH = D // H
FF = 256
KK = 8
TEMP = 0.1
_SQRT_DH = np.sqrt(DH).astype(np.float32)
_SQRT_HALF = np.sqrt(0.5).astype(np.float32)


def _ln(x, g, b, eps=1e-5):
    m = jnp.mean(x, axis=-1, keepdims=True)
    v = jnp.mean((x - m) ** 2, axis=-1, keepdims=True)
    return (x - m) / jnp.sqrt(v + eps) * g + b


def _softmax(x):
    m = jnp.max(x, axis=-1, keepdims=True)
    e = jnp.exp(x - m)
    return e / jnp.sum(e, axis=-1, keepdims=True)


def _gelu(x):
    return 0.5 * x * (1.0 + lax.erf(x * _SQRT_HALF))


def _bdot(a, w_bf16):
    return jnp.dot(a.astype(jnp.bfloat16), w_bf16,
                   preferred_element_type=jnp.float32)


def _encoder_kernel(L, SB, n_layers, h_ref, *refs):
    it = iter(refs)
    layers = [[next(it) for _ in range(12)] for _ in range(n_layers)]
    (wp_ref, bp_ref, gp_ref, bpl_ref, wv1_ref, bv1_ref, gv_ref, bvl_ref,
     wv2_ref, bv2_ref, swv_ref, oswv_ref) = [next(it) for _ in range(12)]
    out_ref = next(it)

    x = h_ref[...]
    for (wqkv_ref, bqkv_ref, wo_ref, bo_ref, g1_ref, b1_ref, w1_ref,
         bff1_ref, w2_ref, bff2_ref, g2_ref, b2_ref) in layers:
        qkv = _bdot(x, wqkv_ref[...]) + bqkv_ref[...]
        qb = qkv[:, :D].astype(jnp.bfloat16)
        kb = qkv[:, D:2 * D].astype(jnp.bfloat16)
        vb = qkv[:, 2 * D:].astype(jnp.bfloat16)
        outs = []
        for s in range(SB):
            r0 = s * L
            heads = []
            for hh in range(H):
                sl = slice(hh * DH, (hh + 1) * DH)
                sc = lax.dot_general(qb[r0:r0 + L, sl], kb[r0:r0 + L, sl],
                                     (((1,), (1,)), ((), ())),
                                     preferred_element_type=jnp.float32)
                a = _softmax(sc / _SQRT_DH)
                heads.append(jnp.dot(a.astype(jnp.bfloat16), vb[r0:r0 + L, sl],
                                     preferred_element_type=jnp.float32))
            outs.append(jnp.concatenate(heads, axis=1))
        o = jnp.concatenate(outs, axis=0)
        o = _bdot(o, wo_ref[...]) + bo_ref[...]
        x = _ln(x + o, g1_ref[...], b1_ref[...])
        h = _bdot(x, w1_ref[...]) + bff1_ref[...]
        h = _gelu(h)
        h = _bdot(h, w2_ref[...]) + bff2_ref[...]
        x = _ln(x + h, g2_ref[...], b2_ref[...])

    cls = jnp.concatenate([x[s * L:s * L + 1, :] for s in range(SB)], axis=0)
    qr = _ln(_bdot(cls, wp_ref[...]) + bp_ref[...], gp_ref[...], bpl_ref[...])

    xm = jnp.concatenate(
        [jnp.mean(h_ref[s * L + 1:(s + 1) * L, :], axis=0, keepdims=True)
         for s in range(SB)], axis=0)
    hv = _bdot(xm, wv1_ref[...]) + bv1_ref[...]
    hv = _gelu(_ln(hv, gv_ref[...], bvl_ref[...]))
    qv = _bdot(hv, wv2_ref[...]) + bv2_ref[...]

    out_ref[...] = swv_ref[...] * qr + oswv_ref[...] * qv


def _topk_kernel(n_keys, qk_ref, mkt_ref, w_ref, ti_ref, ts_scr):
    qk = qk_ref[...]
    mkt = mkt_ref[...]
    dot = jnp.dot(qk, mkt, preferred_element_type=jnp.float32,
                  precision=lax.Precision.HIGHEST)
    nq = jnp.sum(qk * qk, axis=1, keepdims=True)
    nk = jnp.sum(mkt * mkt, axis=0, keepdims=True)
    d = nq + nk - 2.0 * dot
    sim = -d / TEMP
    iota = lax.broadcasted_iota(jnp.int32, sim.shape, 1)
    for j in range(KK):
        m = jnp.max(sim, axis=1, keepdims=True)
        cand = jnp.where(sim == m, iota, jnp.int32(2 ** 30))
        idx = jnp.min(cand, axis=1, keepdims=True)
        ts_scr[:, j:j + 1] = m
        ti_ref[:, j:j + 1] = idx
        sim = jnp.where(iota == idx, -jnp.inf, sim)
    w_ref[...] = _softmax(ts_scr[...])


def _sc_gather(memory_values, idx):
    n_rows, sl, dd = memory_values.shape
    b = idx.shape[0]
    info = plsc.get_sparse_core_info()
    nc, ns = info.num_cores, info.num_subcores
    nw = nc * ns
    b_per_w = b // nw
    mesh = plsc.VectorSubcoreMesh(core_axis_name="c", subcore_axis_name="s")

    @functools.partial(
        pl.kernel, mesh=mesh,
        out_type=jax.ShapeDtypeStruct((b, sl, dd), jnp.float32),
        scratch_types=[
            pltpu.VMEM((b_per_w,), jnp.int32),
            pltpu.VMEM((b_per_w, sl, dd), jnp.float32),
            pltpu.SemaphoreType.DMA,
        ],
    )
    def gather(mv_hbm, idx_hbm, out_hbm, idx_v, rows_v, sem):
        wid = lax.axis_index("s") * nc + lax.axis_index("c")
        base = wid * b_per_w
        pltpu.sync_copy(idx_hbm.at[pl.ds(base, b_per_w)], idx_v)
        pltpu.async_copy(mv_hbm.at[idx_v], rows_v, sem).wait()
        pltpu.sync_copy(rows_v, out_hbm.at[pl.ds(base, b_per_w)])

    return gather(memory_values, idx)


def kernel(query, memory_keys, memory_values, params):
    p = params
    b, lq, _ = query.shape
    L = lq + 1
    n_layers = len(p['layers'])

    cls = jnp.broadcast_to(p['cls'], (b, 1, D))
    h0 = jnp.concatenate([cls, query], axis=1)

    def wt(w):
        return w.T.astype(jnp.bfloat16)

    def row(v):
        return v.reshape(1, -1)

    wrefs = []
    for lp in p['layers']:
        wrefs += [wt(lp['Wqkv']), row(lp['bqkv']), wt(lp['Wo']), row(lp['bo']),
                  row(lp['g1']), row(lp['b1']), wt(lp['W1']), row(lp['bff1']),
                  wt(lp['W2']), row(lp['bff2']), row(lp['g2']), row(lp['b2'])]
    sw = p['sw']
    swv = jnp.broadcast_to(sw.reshape(1, 1), (1, D)).astype(jnp.float32)
    oswv = jnp.broadcast_to((1.0 - sw).reshape(1, 1), (1, D)).astype(jnp.float32)
    wrefs += [wt(p['Wp']), row(p['bp']), row(p['gp']), row(p['bpl']),
              wt(p['Wv1']), row(p['bv1']), row(p['gv']), row(p['bvl']),
              wt(p['Wv2']), row(p['bv2']), swv, oswv]

    const_spec = [pl.BlockSpec(x.shape, lambda i, nd=x.ndim: (0,) * nd)
                  for x in wrefs]
    SB = 16
    hflat = h0.reshape(b * L, D)
    qk = pl.pallas_call(
        functools.partial(_encoder_kernel, L, SB, n_layers),
        grid=(b // SB,),
        in_specs=[pl.BlockSpec((SB * L, D), lambda i: (i, 0))] + const_spec,
        out_specs=pl.BlockSpec((SB, D), lambda i: (i, 0)),
        out_shape=jax.ShapeDtypeStruct((b, D), jnp.float32),
    )(hflat, *wrefs)

    n_keys = memory_keys.shape[0]
    w, ti = pl.pallas_call(
        functools.partial(_topk_kernel, n_keys),
        out_shape=[jax.ShapeDtypeStruct((b, KK), jnp.float32),
                   jax.ShapeDtypeStruct((b, KK), jnp.int32)],
        scratch_shapes=[pltpu.VMEM((b, KK), jnp.float32)],
    )(qk, memory_keys.T)

    refs_flat = _sc_gather(memory_values, ti.reshape(b * KK))
    refs = refs_flat.reshape(b, KK, *memory_values.shape[1:])
    return refs, w


# speedup vs baseline: 1.8342x; 1.8322x over previous
import functools

import jax
import jax.numpy as jnp
import numpy as np
from jax import lax
from jax.experimental import pallas as pl
from jax.experimental.pallas import tpu as pltpu
from jax.experimental.pallas import tpu_sc as plsc

D = 128
DR = 128
H = 4
DH = D // H
FF = 256
KK = 8
TEMP = 0.1
_SQRT_DH = np.sqrt(DH).astype(np.float32)
_SQRT_HALF = np.sqrt(0.5).astype(np.float32)


def _ln(x, g, b, eps=1e-5):
    m = jnp.mean(x, axis=-1, keepdims=True)
    v = jnp.mean((x - m) ** 2, axis=-1, keepdims=True)
    return (x - m) / jnp.sqrt(v + eps) * g + b


def _softmax(x):
    m = jnp.max(x, axis=-1, keepdims=True)
    e = jnp.exp(x - m)
    return e / jnp.sum(e, axis=-1, keepdims=True)


def _gelu(x):
    return 0.5 * x * (1.0 + lax.erf(x * _SQRT_HALF))


def _bdot(a, w_bf16):
    return jnp.dot(a.astype(jnp.bfloat16), w_bf16,
                   preferred_element_type=jnp.float32)


def _attn_sample(qs, ks, vs, L):
    qt = jnp.concatenate([qs] * H, axis=0)
    rblk = lax.broadcasted_iota(jnp.int32, (H * L, D), 0) // L
    chead = lax.broadcasted_iota(jnp.int32, (H * L, D), 1) // DH
    qblock = jnp.where(rblk == chead, qt, jnp.bfloat16(0))
    sc = lax.dot_general(qblock, ks, (((1,), (1,)), ((), ())),
                         preferred_element_type=jnp.float32)
    a = _softmax(sc / _SQRT_DH)
    of = jnp.dot(a.astype(jnp.bfloat16), vs,
                 preferred_element_type=jnp.float32)
    ch = lax.broadcasted_iota(jnp.int32, (L, D), 1) // DH
    o = jnp.where(ch == 0, of[:L, :], 0.0)
    for hh in range(1, H):
        o = o + jnp.where(ch == hh, of[hh * L:(hh + 1) * L, :], 0.0)
    return o


def _fused_kernel(L, SB, n_layers, h_ref, mkt_ref, *refs):
    it = iter(refs)
    layers = [[next(it) for _ in range(12)] for _ in range(n_layers)]
    (wp_ref, bp_ref, gp_ref, bpl_ref, wv1_ref, bv1_ref, gv_ref, bvl_ref,
     wv2_ref, bv2_ref, swv_ref, oswv_ref) = [next(it) for _ in range(12)]
    w_ref = next(it)
    ti_ref = next(it)

    x = h_ref[...]
    for (wqkv_ref, bqkv_ref, wo_ref, bo_ref, g1_ref, b1_ref, w1_ref,
         bff1_ref, w2_ref, bff2_ref, g2_ref, b2_ref) in layers:
        qkv = _bdot(x, wqkv_ref[...]) + bqkv_ref[...]
        qb = qkv[:, :D].astype(jnp.bfloat16)
        kb = qkv[:, D:2 * D].astype(jnp.bfloat16)
        vb = qkv[:, 2 * D:].astype(jnp.bfloat16)
        outs = []
        for s in range(SB):
            r0 = s * L
            outs.append(_attn_sample(qb[r0:r0 + L], kb[r0:r0 + L],
                                     vb[r0:r0 + L], L))
        o = jnp.concatenate(outs, axis=0)
        o = _bdot(o, wo_ref[...]) + bo_ref[...]
        x = _ln(x + o, g1_ref[...], b1_ref[...])
        h = _bdot(x, w1_ref[...]) + bff1_ref[...]
        h = _gelu(h)
        h = _bdot(h, w2_ref[...]) + bff2_ref[...]
        x = _ln(x + h, g2_ref[...], b2_ref[...])

    cls = jnp.concatenate([x[s * L:s * L + 1, :] for s in range(SB)], axis=0)
    qr = _ln(_bdot(cls, wp_ref[...]) + bp_ref[...], gp_ref[...], bpl_ref[...])

    xm = jnp.concatenate(
        [jnp.mean(h_ref[s * L + 1:(s + 1) * L, :], axis=0, keepdims=True)
         for s in range(SB)], axis=0)
    hv = _bdot(xm, wv1_ref[...]) + bv1_ref[...]
    hv = _gelu(_ln(hv, gv_ref[...], bvl_ref[...]))
    qv = _bdot(hv, wv2_ref[...]) + bv2_ref[...]

    qk = swv_ref[...] * qr + oswv_ref[...] * qv

    mkt = mkt_ref[...]
    dot = jnp.dot(qk, mkt, preferred_element_type=jnp.float32,
                  precision=lax.Precision.HIGHEST)
    nq = jnp.sum(qk * qk, axis=1, keepdims=True)
    nk = jnp.sum(mkt * mkt, axis=0, keepdims=True)
    d = nq + nk - 2.0 * dot
    sim = -d / TEMP
    iota = lax.broadcasted_iota(jnp.int32, sim.shape, 1)
    ts = []
    for j in range(KK):
        m = jnp.max(sim, axis=1, keepdims=True)
        cand = jnp.where(sim == m, iota, jnp.int32(2 ** 30))
        idx = jnp.min(cand, axis=1, keepdims=True)
        ts.append(m)
        ti_ref[:, j:j + 1] = idx
        sim = jnp.where(iota == idx, -jnp.inf, sim)
    w_ref[...] = _softmax(jnp.concatenate(ts, axis=1))


def _sc_gather(memory_values, idx):
    n_rows, sl, dd = memory_values.shape
    b = idx.shape[0]
    info = plsc.get_sparse_core_info()
    nc, ns = info.num_cores, info.num_subcores
    nw = nc * ns
    b_per_w = b // nw
    mesh = plsc.VectorSubcoreMesh(core_axis_name="c", subcore_axis_name="s")

    @functools.partial(
        pl.kernel, mesh=mesh,
        out_type=jax.ShapeDtypeStruct((b, sl, dd), jnp.float32),
        scratch_types=[
            pltpu.VMEM((b_per_w,), jnp.int32),
            pltpu.VMEM((b_per_w, sl, dd), jnp.float32),
            pltpu.SemaphoreType.DMA,
        ],
    )
    def gather(mv_hbm, idx_hbm, out_hbm, idx_v, rows_v, sem):
        wid = lax.axis_index("s") * nc + lax.axis_index("c")
        base = wid * b_per_w
        pltpu.sync_copy(idx_hbm.at[pl.ds(base, b_per_w)], idx_v)
        pltpu.async_copy(mv_hbm.at[idx_v], rows_v, sem).wait()
        pltpu.sync_copy(rows_v, out_hbm.at[pl.ds(base, b_per_w)])

    return gather(memory_values, idx)


def kernel(query, memory_keys, memory_values, params):
    p = params
    b, lq, _ = query.shape
    L = lq + 1
    n_layers = len(p['layers'])
    n_keys = memory_keys.shape[0]

    cls = jnp.broadcast_to(p['cls'], (b, 1, D))
    h0 = jnp.concatenate([cls, query], axis=1)

    def wt(w):
        return w.T.astype(jnp.bfloat16)

    def row(v):
        return v.reshape(1, -1)

    wrefs = []
    for lp in p['layers']:
        wrefs += [wt(lp['Wqkv']), row(lp['bqkv']), wt(lp['Wo']), row(lp['bo']),
                  row(lp['g1']), row(lp['b1']), wt(lp['W1']), row(lp['bff1']),
                  wt(lp['W2']), row(lp['bff2']), row(lp['g2']), row(lp['b2'])]
    sw = p['sw']
    swv = jnp.broadcast_to(sw.reshape(1, 1), (1, D)).astype(jnp.float32)
    oswv = jnp.broadcast_to((1.0 - sw).reshape(1, 1), (1, D)).astype(jnp.float32)
    wrefs += [wt(p['Wp']), row(p['bp']), row(p['gp']), row(p['bpl']),
              wt(p['Wv1']), row(p['bv1']), row(p['gv']), row(p['bvl']),
              wt(p['Wv2']), row(p['bv2']), swv, oswv]

    const_spec = [pl.BlockSpec(x.shape, lambda i, nd=x.ndim: (0,) * nd)
                  for x in wrefs]
    SB = 16
    hflat = h0.reshape(b * L, D)
    mkt = memory_keys.T
    w, ti = pl.pallas_call(
        functools.partial(_fused_kernel, L, SB, n_layers),
        grid=(b // SB,),
        in_specs=[pl.BlockSpec((SB * L, D), lambda i: (i, 0)),
                  pl.BlockSpec((D, n_keys), lambda i: (0, 0))] + const_spec,
        out_specs=[pl.BlockSpec((SB, KK), lambda i: (i, 0)),
                   pl.BlockSpec((SB, KK), lambda i: (i, 0))],
        out_shape=[jax.ShapeDtypeStruct((b, KK), jnp.float32),
                   jax.ShapeDtypeStruct((b, KK), jnp.int32)],
    )(hflat, mkt, *wrefs)

    refs_flat = _sc_gather(memory_values, ti.reshape(b * KK))
    refs = refs_flat.reshape(b, KK, *memory_values.shape[1:])
    return refs, w


# speedup vs baseline: 2.0409x; 1.1127x over previous
import functools

import jax
import jax.numpy as jnp
import numpy as np
from jax import lax
from jax.experimental import pallas as pl
from jax.experimental.pallas import tpu as pltpu
from jax.experimental.pallas import tpu_sc as plsc

D = 128
DR = 128
H = 4
DH = D // H
FF = 256
KK = 8
TEMP = 0.1
_SQRT_DH = np.sqrt(DH).astype(np.float32)
_SQRT_HALF = np.sqrt(0.5).astype(np.float32)


def _ln(x, g, b, eps=1e-5):
    m = jnp.mean(x, axis=-1, keepdims=True)
    v = jnp.mean((x - m) ** 2, axis=-1, keepdims=True)
    return (x - m) / jnp.sqrt(v + eps) * g + b


def _softmax(x):
    m = jnp.max(x, axis=-1, keepdims=True)
    e = jnp.exp(x - m)
    return e / jnp.sum(e, axis=-1, keepdims=True)


def _gelu(x):
    return 0.5 * x * (1.0 + lax.erf(x * _SQRT_HALF))


def _bdot_nt(a, w_ref):
    return lax.dot_general(a.astype(jnp.bfloat16),
                           w_ref[...].astype(jnp.bfloat16),
                           (((1,), (1,)), ((), ())),
                           preferred_element_type=jnp.float32)


def _attn_sample(qs, ks, vs, L):
    qt = jnp.concatenate([qs] * H, axis=0)
    rblk = lax.broadcasted_iota(jnp.int32, (H * L, D), 0) // L
    chead = lax.broadcasted_iota(jnp.int32, (H * L, D), 1) // DH
    qblock = jnp.where(rblk == chead, qt, jnp.bfloat16(0))
    sc = lax.dot_general(qblock, ks, (((1,), (1,)), ((), ())),
                         preferred_element_type=jnp.float32)
    a = _softmax(sc / _SQRT_DH)
    of = jnp.dot(a.astype(jnp.bfloat16), vs,
                 preferred_element_type=jnp.float32)
    ch = lax.broadcasted_iota(jnp.int32, (L, D), 1) // DH
    o = jnp.where(ch == 0, of[:L, :], 0.0)
    for hh in range(1, H):
        o = o + jnp.where(ch == hh, of[hh * L:(hh + 1) * L, :], 0.0)
    return o


def _fused_kernel(L, SB, n_layers, q_ref, mk_ref, nk_ref, cls_ref, *refs):
    it = iter(refs)
    layers = [[next(it) for _ in range(12)] for _ in range(n_layers)]
    (wp_ref, bp_ref, gp_ref, bpl_ref, wv1_ref, bv1_ref, gv_ref, bvl_ref,
     wv2_ref, bv2_ref, swv_ref, oswv_ref) = [next(it) for _ in range(12)]
    w_ref = next(it)
    ti_ref = next(it)

    lq = L - 1
    q_all = q_ref[...]
    cls_row = cls_ref[...]
    pieces = []
    for s in range(SB):
        pieces.append(cls_row)
        pieces.append(q_all[s * lq:(s + 1) * lq])
    x = jnp.concatenate(pieces, axis=0)

    for (wqkv_ref, bqkv_ref, wo_ref, bo_ref, g1_ref, b1_ref, w1_ref,
         bff1_ref, w2_ref, bff2_ref, g2_ref, b2_ref) in layers:
        qkv = _bdot_nt(x, wqkv_ref) + bqkv_ref[...]
        qb = qkv[:, :D].astype(jnp.bfloat16)
        kb = qkv[:, D:2 * D].astype(jnp.bfloat16)
        vb = qkv[:, 2 * D:].astype(jnp.bfloat16)
        outs = []
        for s in range(SB):
            r0 = s * L
            outs.append(_attn_sample(qb[r0:r0 + L], kb[r0:r0 + L],
                                     vb[r0:r0 + L], L))
        o = jnp.concatenate(outs, axis=0)
        o = _bdot_nt(o, wo_ref) + bo_ref[...]
        x = _ln(x + o, g1_ref[...], b1_ref[...])
        h = _bdot_nt(x, w1_ref) + bff1_ref[...]
        h = _gelu(h)
        h = _bdot_nt(h, w2_ref) + bff2_ref[...]
        x = _ln(x + h, g2_ref[...], b2_ref[...])

    cls = jnp.concatenate([x[s * L:s * L + 1, :] for s in range(SB)], axis=0)
    qr = _ln(_bdot_nt(cls, wp_ref) + bp_ref[...], gp_ref[...], bpl_ref[...])

    xm = jnp.concatenate(
        [jnp.mean(q_all[s * lq:(s + 1) * lq], axis=0, keepdims=True)
         for s in range(SB)], axis=0)
    hv = _bdot_nt(xm, wv1_ref) + bv1_ref[...]
    hv = _gelu(_ln(hv, gv_ref[...], bvl_ref[...]))
    qv = _bdot_nt(hv, wv2_ref) + bv2_ref[...]

    qk = swv_ref[...] * qr + oswv_ref[...] * qv

    dot = lax.dot_general(qk, mk_ref[...], (((1,), (1,)), ((), ())),
                          preferred_element_type=jnp.float32,
                          precision=lax.Precision.HIGHEST)
    nq = jnp.sum(qk * qk, axis=1, keepdims=True)
    d = nq + nk_ref[...] - 2.0 * dot
    sim = -d / TEMP
    iota = lax.broadcasted_iota(jnp.int32, sim.shape, 1)
    ts = []
    for j in range(KK):
        m = jnp.max(sim, axis=1, keepdims=True)
        cand = jnp.where(sim == m, iota, jnp.int32(2 ** 30))
        idx = jnp.min(cand, axis=1, keepdims=True)
        ts.append(m)
        ti_ref[:, j:j + 1] = idx
        sim = jnp.where(iota == idx, -jnp.inf, sim)
    w_ref[...] = _softmax(jnp.concatenate(ts, axis=1))


def _sc_gather(memory_values, idx):
    n_rows, sl, dd = memory_values.shape
    b = idx.shape[0]
    info = plsc.get_sparse_core_info()
    nc, ns = info.num_cores, info.num_subcores
    nw = nc * ns
    b_per_w = b // nw
    mesh = plsc.VectorSubcoreMesh(core_axis_name="c", subcore_axis_name="s")

    @functools.partial(
        pl.kernel, mesh=mesh,
        out_type=jax.ShapeDtypeStruct((b, sl, dd), jnp.float32),
        scratch_types=[
            pltpu.VMEM((b_per_w,), jnp.int32),
            pltpu.VMEM((b_per_w, sl, dd), jnp.float32),
            pltpu.SemaphoreType.DMA,
        ],
    )
    def gather(mv_hbm, idx_hbm, out_hbm, idx_v, rows_v, sem):
        wid = lax.axis_index("s") * nc + lax.axis_index("c")
        base = wid * b_per_w
        pltpu.sync_copy(idx_hbm.at[pl.ds(base, b_per_w)], idx_v)
        pltpu.async_copy(mv_hbm.at[idx_v], rows_v, sem).wait()
        pltpu.sync_copy(rows_v, out_hbm.at[pl.ds(base, b_per_w)])

    return gather(memory_values, idx)


def kernel(query, memory_keys, memory_values, params):
    p = params
    b, lq, _ = query.shape
    L = lq + 1
    n_layers = len(p['layers'])
    n_keys = memory_keys.shape[0]

    def row(v):
        return v.reshape(1, -1)

    wrefs = []
    for lp in p['layers']:
        wrefs += [lp['Wqkv'], row(lp['bqkv']), lp['Wo'], row(lp['bo']),
                  row(lp['g1']), row(lp['b1']), lp['W1'], row(lp['bff1']),
                  lp['W2'], row(lp['bff2']), row(lp['g2']), row(lp['b2'])]
    sw = p['sw']
    swv = jnp.broadcast_to(sw.reshape(1, 1), (1, D)).astype(jnp.float32)
    oswv = jnp.broadcast_to((1.0 - sw).reshape(1, 1), (1, D)).astype(jnp.float32)
    wrefs += [p['Wp'], row(p['bp']), row(p['gp']), row(p['bpl']),
              p['Wv1'], row(p['bv1']), row(p['gv']), row(p['bvl']),
              p['Wv2'], row(p['bv2']), swv, oswv]

    const_spec = [pl.BlockSpec(x.shape, lambda i, nd=x.ndim: (0,) * nd)
                  for x in wrefs]
    SB = 16
    qflat = query.reshape(b * lq, D)
    nk = row(jnp.sum(memory_keys * memory_keys, axis=1))
    cls2 = p['cls'].reshape(1, D)
    w, ti = pl.pallas_call(
        functools.partial(_fused_kernel, L, SB, n_layers),
        grid=(b // SB,),
        in_specs=[pl.BlockSpec((SB * lq, D), lambda i: (i, 0)),
                  pl.BlockSpec((n_keys, D), lambda i: (0, 0)),
                  pl.BlockSpec((1, n_keys), lambda i: (0, 0)),
                  pl.BlockSpec((1, D), lambda i: (0, 0))] + const_spec,
        out_specs=[pl.BlockSpec((SB, KK), lambda i: (i, 0)),
                   pl.BlockSpec((SB, KK), lambda i: (i, 0))],
        out_shape=[jax.ShapeDtypeStruct((b, KK), jnp.float32),
                   jax.ShapeDtypeStruct((b, KK), jnp.int32)],
    )(qflat, memory_keys, nk, cls2, *wrefs)

    refs_flat = _sc_gather(memory_values, ti.reshape(b * KK))
    refs = refs_flat.reshape(b, KK, *memory_values.shape[1:])
    return refs, w


# speedup vs baseline: 2.4564x; 1.2035x over previous
import functools

import jax
import jax.numpy as jnp
import numpy as np
from jax import lax
from jax.experimental import pallas as pl
from jax.experimental.pallas import tpu as pltpu
from jax.experimental.pallas import tpu_sc as plsc

D = 128
DR = 128
H = 4
DH = D // H
FF = 256
KK = 8
TEMP = 0.1
_SQRT_DH = np.sqrt(DH).astype(np.float32)
_SQRT_HALF = np.sqrt(0.5).astype(np.float32)


def _ln(x, g, b, eps=1e-5):
    m = jnp.mean(x, axis=-1, keepdims=True)
    v = jnp.mean((x - m) ** 2, axis=-1, keepdims=True)
    return (x - m) / jnp.sqrt(v + eps) * g + b


def _softmax(x):
    m = jnp.max(x, axis=-1, keepdims=True)
    e = jnp.exp(x - m)
    return e / jnp.sum(e, axis=-1, keepdims=True)


def _gelu(x):
    return 0.5 * x * (1.0 + lax.erf(x * _SQRT_HALF))


def _bdot_nt(a, w_ref):
    return lax.dot_general(a.astype(jnp.bfloat16),
                           w_ref[...].astype(jnp.bfloat16),
                           (((1,), (1,)), ((), ())),
                           preferred_element_type=jnp.float32)


def _attn_sample(qs, ks, vs, L):
    qt = jnp.concatenate([qs] * H, axis=0)
    rblk = lax.broadcasted_iota(jnp.int32, (H * L, D), 0) // L
    chead = lax.broadcasted_iota(jnp.int32, (H * L, D), 1) // DH
    qblock = jnp.where(rblk == chead, qt, jnp.bfloat16(0))
    sc = lax.dot_general(qblock, ks, (((1,), (1,)), ((), ())),
                         preferred_element_type=jnp.float32)
    a = _softmax(sc / _SQRT_DH)
    of = jnp.dot(a.astype(jnp.bfloat16), vs,
                 preferred_element_type=jnp.float32)
    ch = lax.broadcasted_iota(jnp.int32, (L, D), 1) // DH
    o = jnp.where(ch == 0, of[:L, :], 0.0)
    for hh in range(1, H):
        o = o + jnp.where(ch == hh, of[hh * L:(hh + 1) * L, :], 0.0)
    return o


def _fused_kernel(L, SB, n_layers, q_ref, mk_ref, nk_ref, cls_ref, *refs):
    it = iter(refs)
    layers = [[next(it) for _ in range(12)] for _ in range(n_layers)]
    (wp_ref, bp_ref, gp_ref, bpl_ref, wv1_ref, bv1_ref, gv_ref, bvl_ref,
     wv2_ref, bv2_ref, swv_ref, oswv_ref) = [next(it) for _ in range(12)]
    w_ref = next(it)
    ti_ref = next(it)

    lq = L - 1
    q_all = q_ref[...]
    cls_row = cls_ref[...]
    pieces = []
    for s in range(SB):
        pieces.append(cls_row)
        pieces.append(q_all[s * lq:(s + 1) * lq])
    x = jnp.concatenate(pieces, axis=0)

    for (wqkv_ref, bqkv_ref, wo_ref, bo_ref, g1_ref, b1_ref, w1_ref,
         bff1_ref, w2_ref, bff2_ref, g2_ref, b2_ref) in layers:
        qkv = _bdot_nt(x, wqkv_ref) + bqkv_ref[...]
        qb = qkv[:, :D].astype(jnp.bfloat16)
        kb = qkv[:, D:2 * D].astype(jnp.bfloat16)
        vb = qkv[:, 2 * D:].astype(jnp.bfloat16)
        outs = []
        for s in range(SB):
            r0 = s * L
            outs.append(_attn_sample(qb[r0:r0 + L], kb[r0:r0 + L],
                                     vb[r0:r0 + L], L))
        o = jnp.concatenate(outs, axis=0)
        o = _bdot_nt(o, wo_ref) + bo_ref[...]
        x = _ln(x + o, g1_ref[...], b1_ref[...])
        h = _bdot_nt(x, w1_ref) + bff1_ref[...]
        h = _gelu(h)
        h = _bdot_nt(h, w2_ref) + bff2_ref[...]
        x = _ln(x + h, g2_ref[...], b2_ref[...])

    cls = jnp.concatenate([x[s * L:s * L + 1, :] for s in range(SB)], axis=0)
    qr = _ln(_bdot_nt(cls, wp_ref) + bp_ref[...], gp_ref[...], bpl_ref[...])

    xm = jnp.concatenate(
        [jnp.mean(q_all[s * lq:(s + 1) * lq], axis=0, keepdims=True)
         for s in range(SB)], axis=0)
    hv = _bdot_nt(xm, wv1_ref) + bv1_ref[...]
    hv = _gelu(_ln(hv, gv_ref[...], bvl_ref[...]))
    qv = _bdot_nt(hv, wv2_ref) + bv2_ref[...]

    qk = swv_ref[...] * qr + oswv_ref[...] * qv

    dot = lax.dot_general(qk, mk_ref[...], (((1,), (1,)), ((), ())),
                          preferred_element_type=jnp.float32,
                          precision=lax.Precision.HIGHEST)
    nq = jnp.sum(qk * qk, axis=1, keepdims=True)
    d = nq + nk_ref[...] - 2.0 * dot
    sim = -d / TEMP
    iota = lax.broadcasted_iota(jnp.int32, sim.shape, 1)
    ts = []
    for j in range(KK):
        m = jnp.max(sim, axis=1, keepdims=True)
        cand = jnp.where(sim == m, iota, jnp.int32(2 ** 30))
        idx = jnp.min(cand, axis=1, keepdims=True)
        ts.append(m)
        ti_ref[:, j:j + 1] = idx
        sim = jnp.where(iota == idx, -jnp.inf, sim)
    w_ref[...] = _softmax(jnp.concatenate(ts, axis=1))


def _sc_gather(memory_values, idx):
    n_rows, sl, dd = memory_values.shape
    b = idx.shape[0]
    info = plsc.get_sparse_core_info()
    nc, ns = info.num_cores, info.num_subcores
    nw = nc * ns
    b_per_w = b // nw
    mesh = plsc.VectorSubcoreMesh(core_axis_name="c", subcore_axis_name="s")

    @functools.partial(
        pl.kernel, mesh=mesh,
        out_type=jax.ShapeDtypeStruct((b, sl, dd), jnp.float32),
        scratch_types=[
            pltpu.VMEM((b_per_w,), jnp.int32),
            pltpu.VMEM((b_per_w, sl, dd), jnp.float32),
            pltpu.SemaphoreType.DMA,
        ],
    )
    def gather(mv_hbm, idx_hbm, out_hbm, idx_v, rows_v, sem):
        wid = lax.axis_index("s") * nc + lax.axis_index("c")
        base = wid * b_per_w
        pltpu.sync_copy(idx_hbm.at[pl.ds(base, b_per_w)], idx_v)
        pltpu.async_copy(mv_hbm.at[idx_v], rows_v, sem).wait()
        pltpu.sync_copy(rows_v, out_hbm.at[pl.ds(base, b_per_w)])

    return gather(memory_values, idx)


def kernel(query, memory_keys, memory_values, params):
    p = params
    b, lq, _ = query.shape
    L = lq + 1
    n_layers = len(p['layers'])
    n_keys = memory_keys.shape[0]

    def row(v):
        return v.reshape(1, -1)

    wrefs = []
    for lp in p['layers']:
        wrefs += [lp['Wqkv'], row(lp['bqkv']), lp['Wo'], row(lp['bo']),
                  row(lp['g1']), row(lp['b1']), lp['W1'], row(lp['bff1']),
                  lp['W2'], row(lp['bff2']), row(lp['g2']), row(lp['b2'])]
    sw = p['sw']
    swv = jnp.broadcast_to(sw.reshape(1, 1), (1, D)).astype(jnp.float32)
    oswv = jnp.broadcast_to((1.0 - sw).reshape(1, 1), (1, D)).astype(jnp.float32)
    wrefs += [p['Wp'], row(p['bp']), row(p['gp']), row(p['bpl']),
              p['Wv1'], row(p['bv1']), row(p['gv']), row(p['bvl']),
              p['Wv2'], row(p['bv2']), swv, oswv]

    const_spec = [pl.BlockSpec(x.shape, lambda i, nd=x.ndim: (0,) * nd)
                  for x in wrefs]
    SB = 32
    qflat = query.reshape(b * lq, D)
    nk = row(jnp.sum(memory_keys * memory_keys, axis=1))
    cls2 = p['cls'].reshape(1, D)
    w, ti = pl.pallas_call(
        functools.partial(_fused_kernel, L, SB, n_layers),
        grid=(b // SB,),
        in_specs=[pl.BlockSpec((SB * lq, D), lambda i: (i, 0)),
                  pl.BlockSpec((n_keys, D), lambda i: (0, 0)),
                  pl.BlockSpec((1, n_keys), lambda i: (0, 0)),
                  pl.BlockSpec((1, D), lambda i: (0, 0))] + const_spec,
        out_specs=[pl.BlockSpec((SB, KK), lambda i: (i, 0)),
                   pl.BlockSpec((SB, KK), lambda i: (i, 0))],
        out_shape=[jax.ShapeDtypeStruct((b, KK), jnp.float32),
                   jax.ShapeDtypeStruct((b, KK), jnp.int32)],
    )(qflat, memory_keys, nk, cls2, *wrefs)

    refs_flat = _sc_gather(memory_values, ti.reshape(b * KK))
    refs = refs_flat.reshape(b, KK, *memory_values.shape[1:])
    return refs, w


# speedup vs baseline: 2.4930x; 1.0149x over previous
import functools

import jax
import jax.numpy as jnp
import numpy as np
from jax import lax
from jax.experimental import pallas as pl
from jax.experimental.pallas import tpu as pltpu
from jax.experimental.pallas import tpu_sc as plsc

D = 128
DR = 128
H = 4
DH = D // H
FF = 256
KK = 8
TEMP = 0.1
_SQRT_DH = np.sqrt(DH).astype(np.float32)
_SQRT_HALF = np.sqrt(0.5).astype(np.float32)


def _ln(x, g, b, eps=1e-5):
    m = jnp.mean(x, axis=-1, keepdims=True)
    v = jnp.mean((x - m) ** 2, axis=-1, keepdims=True)
    return (x - m) / jnp.sqrt(v + eps) * g + b


def _softmax(x):
    m = jnp.max(x, axis=-1, keepdims=True)
    e = jnp.exp(x - m)
    return e / jnp.sum(e, axis=-1, keepdims=True)


def _gelu(x):
    return 0.5 * x * (1.0 + lax.erf(x * _SQRT_HALF))


def _bdot_nt(a, w_ref):
    return lax.dot_general(a.astype(jnp.bfloat16),
                           w_ref[...].astype(jnp.bfloat16),
                           (((1,), (1,)), ((), ())),
                           preferred_element_type=jnp.float32)


def _attn_sample(qs, ks, vs, Lp, L):
    qt = jnp.concatenate([qs] * H, axis=0)
    rblk = lax.broadcasted_iota(jnp.int32, (H * Lp, D), 0) // Lp
    chead = lax.broadcasted_iota(jnp.int32, (H * Lp, D), 1) // DH
    qblock = jnp.where(rblk == chead, qt, jnp.bfloat16(0))
    sc = lax.dot_general(qblock, ks, (((1,), (1,)), ((), ())),
                         preferred_element_type=jnp.float32)
    col = lax.broadcasted_iota(jnp.int32, (H * Lp, Lp), 1)
    a = _softmax(jnp.where(col < L, sc / _SQRT_DH, -jnp.inf))
    of = jnp.dot(a.astype(jnp.bfloat16), vs,
                 preferred_element_type=jnp.float32)
    ch = lax.broadcasted_iota(jnp.int32, (Lp, D), 1) // DH
    o = jnp.where(ch == 0, of[:Lp, :], 0.0)
    for hh in range(1, H):
        o = o + jnp.where(ch == hh, of[hh * Lp:(hh + 1) * Lp, :], 0.0)
    return o


def _fused_kernel(Lp, L, SB, n_layers, q_ref, mk_ref, nk_ref, *refs):
    it = iter(refs)
    layers = [[next(it) for _ in range(12)] for _ in range(n_layers)]
    (wp_ref, bp_ref, gp_ref, bpl_ref, wv1_ref, bv1_ref, gv_ref, bvl_ref,
     wv2_ref, bv2_ref, swv_ref, oswv_ref) = [next(it) for _ in range(12)]
    w_ref = next(it)
    ti_ref = next(it)

    lq = L - 1
    x = q_ref[...]

    for (wqkv_ref, bqkv_ref, wo_ref, bo_ref, g1_ref, b1_ref, w1_ref,
         bff1_ref, w2_ref, bff2_ref, g2_ref, b2_ref) in layers:
        qkv = _bdot_nt(x, wqkv_ref) + bqkv_ref[...]
        qb = qkv[:, :D].astype(jnp.bfloat16)
        kb = qkv[:, D:2 * D].astype(jnp.bfloat16)
        vb = qkv[:, 2 * D:].astype(jnp.bfloat16)
        outs = []
        for s in range(SB):
            r0 = s * Lp
            outs.append(_attn_sample(qb[r0:r0 + Lp], kb[r0:r0 + Lp],
                                     vb[r0:r0 + Lp], Lp, L))
        o = jnp.concatenate(outs, axis=0)
        o = _bdot_nt(o, wo_ref) + bo_ref[...]
        x = _ln(x + o, g1_ref[...], b1_ref[...])
        h = _bdot_nt(x, w1_ref) + bff1_ref[...]
        h = _gelu(h)
        h = _bdot_nt(h, w2_ref) + bff2_ref[...]
        x = _ln(x + h, g2_ref[...], b2_ref[...])

    cls = jnp.concatenate([x[s * Lp:s * Lp + 1, :] for s in range(SB)], axis=0)
    qr = _ln(_bdot_nt(cls, wp_ref) + bp_ref[...], gp_ref[...], bpl_ref[...])

    xm = jnp.concatenate(
        [jnp.mean(q_ref[s * Lp + 1:s * Lp + 1 + lq], axis=0, keepdims=True)
         for s in range(SB)], axis=0)
    hv = _bdot_nt(xm, wv1_ref) + bv1_ref[...]
    hv = _gelu(_ln(hv, gv_ref[...], bvl_ref[...]))
    qv = _bdot_nt(hv, wv2_ref) + bv2_ref[...]

    qk = swv_ref[...] * qr + oswv_ref[...] * qv

    dot = lax.dot_general(qk, mk_ref[...], (((1,), (1,)), ((), ())),
                          preferred_element_type=jnp.float32,
                          precision=lax.Precision.HIGHEST)
    nq = jnp.sum(qk * qk, axis=1, keepdims=True)
    d = nq + nk_ref[...] - 2.0 * dot
    sim = -d / TEMP
    iota = lax.broadcasted_iota(jnp.int32, sim.shape, 1)
    ts = []
    for j in range(KK):
        m = jnp.max(sim, axis=1, keepdims=True)
        cand = jnp.where(sim == m, iota, jnp.int32(2 ** 30))
        idx = jnp.min(cand, axis=1, keepdims=True)
        ts.append(m)
        ti_ref[:, j:j + 1] = idx
        sim = jnp.where(iota == idx, -jnp.inf, sim)
    w_ref[...] = _softmax(jnp.concatenate(ts, axis=1))


def _sc_gather(memory_values, idx):
    n_rows, sl, dd = memory_values.shape
    b = idx.shape[0]
    info = plsc.get_sparse_core_info()
    nc, ns = info.num_cores, info.num_subcores
    nw = nc * ns
    b_per_w = b // nw
    mesh = plsc.VectorSubcoreMesh(core_axis_name="c", subcore_axis_name="s")

    @functools.partial(
        pl.kernel, mesh=mesh,
        out_type=jax.ShapeDtypeStruct((b, sl, dd), jnp.float32),
        scratch_types=[
            pltpu.VMEM((b_per_w,), jnp.int32),
            pltpu.VMEM((b_per_w, sl, dd), jnp.float32),
            pltpu.SemaphoreType.DMA,
        ],
    )
    def gather(mv_hbm, idx_hbm, out_hbm, idx_v, rows_v, sem):
        wid = lax.axis_index("s") * nc + lax.axis_index("c")
        base = wid * b_per_w
        pltpu.sync_copy(idx_hbm.at[pl.ds(base, b_per_w)], idx_v)
        pltpu.async_copy(mv_hbm.at[idx_v], rows_v, sem).wait()
        pltpu.sync_copy(rows_v, out_hbm.at[pl.ds(base, b_per_w)])

    return gather(memory_values, idx)


def kernel(query, memory_keys, memory_values, params):
    p = params
    b, lq, _ = query.shape
    L = lq + 1
    n_layers = len(p['layers'])
    n_keys = memory_keys.shape[0]

    def row(v):
        return v.reshape(1, -1)

    wrefs = []
    for lp in p['layers']:
        wrefs += [lp['Wqkv'], row(lp['bqkv']), lp['Wo'], row(lp['bo']),
                  row(lp['g1']), row(lp['b1']), lp['W1'], row(lp['bff1']),
                  lp['W2'], row(lp['bff2']), row(lp['g2']), row(lp['b2'])]
    sw = p['sw']
    swv = jnp.broadcast_to(sw.reshape(1, 1), (1, D)).astype(jnp.float32)
    oswv = jnp.broadcast_to((1.0 - sw).reshape(1, 1), (1, D)).astype(jnp.float32)
    wrefs += [p['Wp'], row(p['bp']), row(p['gp']), row(p['bpl']),
              p['Wv1'], row(p['bv1']), row(p['gv']), row(p['bvl']),
              p['Wv2'], row(p['bv2']), swv, oswv]

    const_spec = [pl.BlockSpec(x.shape, lambda i, nd=x.ndim: (0,) * nd)
                  for x in wrefs]
    SB = 32
    Lp = L + (-L) % 8
    cls = jnp.broadcast_to(p['cls'], (b, 1, D))
    pad = jnp.zeros((b, Lp - L, D), jnp.float32)
    hp = jnp.concatenate([cls, query, pad], axis=1).reshape(b * Lp, D)
    nk = row(jnp.sum(memory_keys * memory_keys, axis=1))
    w, ti = pl.pallas_call(
        functools.partial(_fused_kernel, Lp, L, SB, n_layers),
        grid=(b // SB,),
        in_specs=[pl.BlockSpec((SB * Lp, D), lambda i: (i, 0)),
                  pl.BlockSpec((n_keys, D), lambda i: (0, 0)),
                  pl.BlockSpec((1, n_keys), lambda i: (0, 0))] + const_spec,
        out_specs=[pl.BlockSpec((SB, KK), lambda i: (i, 0)),
                   pl.BlockSpec((SB, KK), lambda i: (i, 0))],
        out_shape=[jax.ShapeDtypeStruct((b, KK), jnp.float32),
                   jax.ShapeDtypeStruct((b, KK), jnp.int32)],
    )(hp, memory_keys, nk, *wrefs)

    refs_flat = _sc_gather(memory_values, ti.reshape(b * KK))
    refs = refs_flat.reshape(b, KK, *memory_values.shape[1:])
    return refs, w
